# baseline (speedup 1.0000x reference)
import functools

import jax
import jax.numpy as jnp
from jax import lax
from jax.experimental import pallas as pl
from jax.experimental.pallas import tpu as pltpu
from jax.experimental.pallas import tpu_sc as plsc

NBUF = 3
SC_PAIRS = 40


def _sc_pool(x_hbm, o_hbm, inbuf, outbuf, in_sems, out_sems, *, J, H, W, HC,
             n_workers, nc, S, sc_pairs):
    wid = lax.axis_index("s") * nc + lax.axis_index("c")
    n_chunks = H // HC
    units = sc_pairs * n_chunks
    upw_lo = units // n_workers
    rem = units - upw_lo * n_workers
    extra = jnp.where(wid < rem, 1, 0)
    base = wid * upw_lo + jnp.minimum(wid, rem)
    upw = upw_lo + extra

    def in_copy(g, buf):
        u = base + g
        pair = u // n_chunks
        b = pair // S
        s = pair % S
        h0 = (u % n_chunks) * HC
        return pltpu.make_async_copy(
            x_hbm.at[b, pl.ds(s * J, J), pl.ds(h0, HC), :],
            inbuf.at[buf], in_sems.at[buf])

    def out_copy(g, buf):
        u = base + g
        pair = u // n_chunks
        h0 = (u % n_chunks) * HC
        return pltpu.make_async_copy(
            outbuf.at[buf], o_hbm.at[pair, pl.ds(h0, HC), :],
            out_sems.at[buf])

    for g0 in range(NBUF - 1):
        in_copy(g0, g0).start()

    def loop(g, _):
        buf = lax.rem(g, NBUF)
        obuf = lax.rem(g, 2)

        @pl.when(g + NBUF - 1 < upw)
        def _prefetch():
            in_copy(g + NBUF - 1, lax.rem(g + NBUF - 1, NBUF)).start()

        in_copy(g, buf).wait()

        @pl.when(g >= 2)
        def _drain():
            out_copy(g - 2, obuf).wait()

        def body(r, _):
            for cg in range(W // 16):
                c0 = cg * 16
                acc = inbuf[buf, 0, r, pl.ds(c0, 16)]
                for j in range(1, J):
                    acc = jnp.maximum(acc, inbuf[buf, j, r, pl.ds(c0, 16)])
                outbuf[obuf, r, pl.ds(c0, 16)] = acc
            return 0

        lax.fori_loop(0, HC, body, 0, unroll=2)
        out_copy(g, obuf).start()
        return 0

    lax.fori_loop(0, upw, loop, 0)
    out_copy(upw - 2, lax.rem(upw - 2, 2)).wait()
    out_copy(upw - 1, lax.rem(upw - 1, 2)).wait()


def _tc_body(idx_ref, x_ref, o_ref):
    o_ref[...] = jnp.max(x_ref[...], axis=1)


def _merge_body(sc_ref, tc_ref, o_ref):
    p = pl.program_id(0)

    @pl.when(p < SC_PAIRS)
    def _take_sc():
        o_ref[...] = sc_ref[...]

    @pl.when(p >= SC_PAIRS)
    def _take_tc():
        o_ref[...] = tc_ref[...]


def kernel(input, channel_idx_sets):
    B, C, H, W = input.shape
    S, J = channel_idx_sets.shape
    HC = 16
    info = plsc.get_sparse_core_info()
    nc, ns = info.num_cores, info.num_subcores
    n_workers = nc * ns
    n_pairs = B * S
    tc_pairs = n_pairs - SC_PAIRS

    mesh = plsc.VectorSubcoreMesh(core_axis_name="c", subcore_axis_name="s")
    sc_body = functools.partial(
        _sc_pool, J=J, H=H, W=W, HC=HC, n_workers=n_workers, nc=nc, S=S,
        sc_pairs=SC_PAIRS)
    sc_out = pl.kernel(
        sc_body,
        mesh=mesh,
        out_type=jax.ShapeDtypeStruct((SC_PAIRS, H, W), jnp.float32),
        scratch_types=[
            pltpu.VMEM((NBUF, J, HC, W), jnp.float32),
            pltpu.VMEM((2, HC, W), jnp.float32),
            pltpu.SemaphoreType.DMA((NBUF,)),
            pltpu.SemaphoreType.DMA((2,)),
        ],
    )(input)

    grid_spec = pltpu.PrefetchScalarGridSpec(
        num_scalar_prefetch=1,
        grid=(tc_pairs,),
        in_specs=[
            pl.BlockSpec(
                (1, J, H, W),
                lambda p, idx: ((SC_PAIRS + p) // S,
                                idx[(SC_PAIRS + p) % S, 0] // J, 0, 0)),
        ],
        out_specs=pl.BlockSpec((1, H, W), lambda p, idx: (p, 0, 0)),
    )
    tc_out = pl.pallas_call(
        _tc_body,
        grid_spec=grid_spec,
        out_shape=jax.ShapeDtypeStruct((tc_pairs, H, W), jnp.float32),
        compiler_params=pltpu.CompilerParams(
            dimension_semantics=("parallel",),
        ),
    )(channel_idx_sets, input)

    out = pl.pallas_call(
        _merge_body,
        grid=(n_pairs,),
        in_specs=[
            pl.BlockSpec(
                (1, H, W), lambda p: (jnp.minimum(p, SC_PAIRS - 1), 0, 0)),
            pl.BlockSpec(
                (1, H, W), lambda p: (jnp.maximum(p - SC_PAIRS, 0), 0, 0)),
        ],
        out_specs=pl.BlockSpec((1, H, W), lambda p: (p, 0, 0)),
        out_shape=jax.ShapeDtypeStruct((n_pairs, H, W), jnp.float32),
        compiler_params=pltpu.CompilerParams(
            dimension_semantics=("arbitrary",),
        ),
    )(sc_out, tc_out)
    return out.reshape(B, S, H, W)


# speedup vs baseline: 1.4747x; 1.4747x over previous
import functools

import jax
import jax.numpy as jnp
from jax import lax
from jax.experimental import pallas as pl
from jax.experimental.pallas import tpu as pltpu
from jax.experimental.pallas import tpu_sc as plsc

NBUF = 3
SC_PAIRS = 40


def _sc_pool(x_hbm, o_hbm, inbuf, outbuf, in_sems, out_sems, *, J, H, W, HC,
             n_workers, nc, S, sc_pairs):
    wid = lax.axis_index("s") * nc + lax.axis_index("c")
    n_chunks = H // HC
    units = sc_pairs * n_chunks
    upw_lo = units // n_workers
    rem = units - upw_lo * n_workers
    extra = jnp.where(wid < rem, 1, 0)
    base = wid * upw_lo + jnp.minimum(wid, rem)
    upw = upw_lo + extra

    def in_copy(g, buf):
        u = base + g
        pair = u // n_chunks
        b = pair // S
        s = pair % S
        h0 = (u % n_chunks) * HC
        return pltpu.make_async_copy(
            x_hbm.at[b, pl.ds(s * J, J), pl.ds(h0, HC), :],
            inbuf.at[buf], in_sems.at[buf])

    def out_copy(g, buf):
        u = base + g
        pair = u // n_chunks
        h0 = (u % n_chunks) * HC
        return pltpu.make_async_copy(
            outbuf.at[buf], o_hbm.at[pair, pl.ds(h0, HC), :],
            out_sems.at[buf])

    for g0 in range(NBUF - 1):
        in_copy(g0, g0).start()

    def loop(g, _):
        buf = lax.rem(g, NBUF)
        obuf = lax.rem(g, 2)

        @pl.when(g + NBUF - 1 < upw)
        def _prefetch():
            in_copy(g + NBUF - 1, lax.rem(g + NBUF - 1, NBUF)).start()

        in_copy(g, buf).wait()

        @pl.when(g >= 2)
        def _drain():
            out_copy(g - 2, obuf).wait()

        def body(r, _):
            for cg in range(W // 16):
                c0 = cg * 16
                acc = inbuf[buf, 0, r, pl.ds(c0, 16)]
                for j in range(1, J):
                    acc = jnp.maximum(acc, inbuf[buf, j, r, pl.ds(c0, 16)])
                outbuf[obuf, r, pl.ds(c0, 16)] = acc
            return 0

        lax.fori_loop(0, HC, body, 0, unroll=2)
        out_copy(g, obuf).start()
        return 0

    lax.fori_loop(0, upw, loop, 0)
    out_copy(upw - 2, lax.rem(upw - 2, 2)).wait()
    out_copy(upw - 1, lax.rem(upw - 1, 2)).wait()


def _tc_body(idx_ref, x_ref, o_ref):
    o_ref[...] = jnp.max(x_ref[...], axis=1)


def kernel(input, channel_idx_sets):
    B, C, H, W = input.shape
    S, J = channel_idx_sets.shape
    HC = 16
    info = plsc.get_sparse_core_info()
    nc, ns = info.num_cores, info.num_subcores
    n_workers = nc * ns
    n_pairs = B * S
    tc_pairs = n_pairs - SC_PAIRS

    mesh = plsc.VectorSubcoreMesh(core_axis_name="c", subcore_axis_name="s")
    sc_body = functools.partial(
        _sc_pool, J=J, H=H, W=W, HC=HC, n_workers=n_workers, nc=nc, S=S,
        sc_pairs=SC_PAIRS)
    sc_out = pl.kernel(
        sc_body,
        mesh=mesh,
        out_type=jax.ShapeDtypeStruct((n_pairs, H, W), jnp.float32),
        scratch_types=[
            pltpu.VMEM((NBUF, J, HC, W), jnp.float32),
            pltpu.VMEM((2, HC, W), jnp.float32),
            pltpu.SemaphoreType.DMA((NBUF,)),
            pltpu.SemaphoreType.DMA((2,)),
        ],
    )(input)

    grid_spec = pltpu.PrefetchScalarGridSpec(
        num_scalar_prefetch=1,
        grid=(tc_pairs,),
        in_specs=[
            pl.BlockSpec(
                (1, J, H, W),
                lambda p, idx: ((SC_PAIRS + p) // S,
                                idx[(SC_PAIRS + p) % S, 0] // J, 0, 0)),
        ],
        out_specs=pl.BlockSpec((1, H, W), lambda p, idx: (p, 0, 0)),
    )
    tc_out = pl.pallas_call(
        _tc_body,
        grid_spec=grid_spec,
        out_shape=jax.ShapeDtypeStruct((tc_pairs, H, W), jnp.float32),
        compiler_params=pltpu.CompilerParams(
            dimension_semantics=("parallel",),
        ),
    )(channel_idx_sets, input)

    out = lax.dynamic_update_slice(sc_out, tc_out, (SC_PAIRS, 0, 0))
    return out.reshape(B, S, H, W)


# speedup vs baseline: 1.5154x; 1.0276x over previous
import functools

import jax
import jax.numpy as jnp
from jax import lax
from jax.experimental import pallas as pl
from jax.experimental.pallas import tpu as pltpu
from jax.experimental.pallas import tpu_sc as plsc

NBUF = 3
SC_PAIRS = 40


def _sc_pool(x_hbm, o_hbm, inbuf, outbuf, in_sems, out_sems, *, J, H, W, HC,
             n_workers, nc, S, sc_pairs):
    wid = lax.axis_index("s") * nc + lax.axis_index("c")
    n_chunks = H // HC
    units = sc_pairs * n_chunks
    upw_lo = units // n_workers
    rem = units - upw_lo * n_workers
    extra = jnp.where(wid < rem, 1, 0)
    base = wid * upw_lo + jnp.minimum(wid, rem)
    upw = upw_lo + extra

    def in_copy(g, buf):
        u = base + g
        pair = u // n_chunks
        b = pair // S
        s = pair % S
        h0 = (u % n_chunks) * HC
        return pltpu.make_async_copy(
            x_hbm.at[b, pl.ds(s * J, J), pl.ds(h0, HC), :],
            inbuf.at[buf], in_sems.at[buf])

    def out_copy(g, buf):
        u = base + g
        pair = u // n_chunks
        h0 = (u % n_chunks) * HC
        return pltpu.make_async_copy(
            outbuf.at[buf], o_hbm.at[pair, pl.ds(h0, HC), :],
            out_sems.at[buf])

    for g0 in range(NBUF - 1):
        in_copy(g0, g0).start()

    def loop(g, _):
        buf = lax.rem(g, NBUF)
        obuf = lax.rem(g, 2)

        @pl.when(g + NBUF - 1 < upw)
        def _prefetch():
            in_copy(g + NBUF - 1, lax.rem(g + NBUF - 1, NBUF)).start()

        in_copy(g, buf).wait()

        @pl.when(g >= 2)
        def _drain():
            out_copy(g - 2, obuf).wait()

        def body(r, _):
            for cg in range(W // 16):
                c0 = cg * 16
                acc = inbuf[buf, 0, r, pl.ds(c0, 16)]
                for j in range(1, J):
                    acc = jnp.maximum(acc, inbuf[buf, j, r, pl.ds(c0, 16)])
                outbuf[obuf, r, pl.ds(c0, 16)] = acc
            return 0

        lax.fori_loop(0, HC, body, 0, unroll=2)
        out_copy(g, obuf).start()
        return 0

    lax.fori_loop(0, upw, loop, 0)
    out_copy(upw - 2, lax.rem(upw - 2, 2)).wait()
    out_copy(upw - 1, lax.rem(upw - 1, 2)).wait()


def _tc_body(idx_ref, x_ref, o_ref):
    o_ref[...] = jnp.max(x_ref[...], axis=1)


def kernel(input, channel_idx_sets):
    B, C, H, W = input.shape
    S, J = channel_idx_sets.shape
    HC = 16
    info = plsc.get_sparse_core_info()
    nc, ns = info.num_cores, info.num_subcores
    n_workers = nc * ns
    n_pairs = B * S
    tc_pairs = n_pairs - SC_PAIRS

    mesh = plsc.VectorSubcoreMesh(core_axis_name="c", subcore_axis_name="s")
    sc_body = functools.partial(
        _sc_pool, J=J, H=H, W=W, HC=HC, n_workers=n_workers, nc=nc, S=S,
        sc_pairs=SC_PAIRS)
    sc_out = pl.kernel(
        sc_body,
        mesh=mesh,
        out_type=jax.ShapeDtypeStruct((SC_PAIRS, H, W), jnp.float32),
        scratch_types=[
            pltpu.VMEM((NBUF, J, HC, W), jnp.float32),
            pltpu.VMEM((2, HC, W), jnp.float32),
            pltpu.SemaphoreType.DMA((NBUF,)),
            pltpu.SemaphoreType.DMA((2,)),
        ],
    )(input)

    grid_spec = pltpu.PrefetchScalarGridSpec(
        num_scalar_prefetch=1,
        grid=(tc_pairs,),
        in_specs=[
            pl.BlockSpec(
                (1, J, H, W),
                lambda p, idx: ((SC_PAIRS + p) // S,
                                idx[(SC_PAIRS + p) % S, 0] // J, 0, 0)),
        ],
        out_specs=pl.BlockSpec(
            (1, H, W), lambda p, idx: (SC_PAIRS + p, 0, 0)),
    )
    tc_out = pl.pallas_call(
        _tc_body,
        grid_spec=grid_spec,
        out_shape=jax.ShapeDtypeStruct((n_pairs, H, W), jnp.float32),
        compiler_params=pltpu.CompilerParams(
            dimension_semantics=("parallel",),
        ),
    )(channel_idx_sets, input)

    out = lax.dynamic_update_slice(tc_out, sc_out, (0, 0, 0))
    return out.reshape(B, S, H, W)
